# baseline (speedup 1.0000x reference)
import functools

import jax
import jax.numpy as jnp
from jax import lax
from jax.experimental import pallas as pl
from jax.experimental.pallas import tpu as pltpu
from jax.experimental.pallas import tpu_sc as plsc

E = 8
K = 2
T = 2048
H = 2048
F = 2048
TK = T * K

BM = 256
S_PAD = TK + E * BM
NBLK = S_PAD // BM
BF = 1024
NFB = F // BF
BM3 = 256

SC_NC = 2
SC_NS = 16
SC_NW = SC_NC * SC_NS
GCH = 16


def _router_body(x_ref, wr_ref, idx_ref, score_ref):
    x = x_ref[...]
    wr = wr_ref[...]
    logits = lax.dot_general(x, wr, (((1,), (1,)), ((), ())),
                             preferred_element_type=jnp.float32)
    lane = lax.broadcasted_iota(jnp.int32, (T, E), 1)
    m1 = jnp.max(logits, axis=1, keepdims=True)
    i1 = jnp.min(jnp.where(logits == m1, lane, E), axis=1, keepdims=True)
    masked = jnp.where(lane == i1, -jnp.inf, logits)
    m2 = jnp.max(masked, axis=1, keepdims=True)
    i2 = jnp.min(jnp.where(masked == m2, lane, E), axis=1, keepdims=True)
    idx_ref[:, 0:1] = i1
    idx_ref[:, 1:2] = i2
    score_ref[:, 0:1] = jax.nn.sigmoid(m1)
    score_ref[:, 1:2] = jax.nn.sigmoid(m2)


def _router(x, w_router):
    return pl.pallas_call(
        _router_body,
        out_shape=[
            jax.ShapeDtypeStruct((T, K), jnp.int32),
            jax.ShapeDtypeStruct((T, K), jnp.float32),
        ],
    )(x, w_router)


NBUF = 3
AHEAD = 2


@functools.lru_cache(maxsize=None)
def _make_sc_gather(n_rows, table_rows, width, dtype):
    rpw = n_rows // SC_NW
    nch = rpw // GCH
    mesh = plsc.VectorSubcoreMesh(core_axis_name="c", subcore_axis_name="s")

    @functools.partial(
        pl.kernel,
        mesh=mesh,
        out_type=jax.ShapeDtypeStruct((n_rows, width), dtype),
        scratch_types=[
            pltpu.VMEM((rpw,), jnp.int32),
            *[pltpu.VMEM((GCH, width), dtype) for _ in range(NBUF)],
            *[pltpu.SemaphoreType.DMA for _ in range(2 * NBUF)],
        ],
    )
    def gather_k(table_hbm, idx_hbm, out_hbm, idx_all, *bufs_sems):
        bufs = bufs_sems[:NBUF]
        gsem = bufs_sems[NBUF:2 * NBUF]
        ssem = bufs_sems[2 * NBUF:]
        wid = lax.axis_index("s") * SC_NC + lax.axis_index("c")
        base = wid * rpw
        pltpu.sync_copy(idx_hbm.at[pl.ds(base, rpw)], idx_all)

        gs, ss = {}, {}
        for step in range(nch + AHEAD):
            i = step
            if i < nch:
                b = i % NBUF
                if i >= NBUF:
                    ss[i - NBUF].wait()
                gs[i] = pltpu.async_copy(
                    table_hbm.at[idx_all.at[pl.ds(i * GCH, GCH)]],
                    bufs[b], gsem[b])
            j = step - AHEAD
            if j >= 0:
                gs[j].wait()
                ss[j] = pltpu.async_copy(
                    bufs[j % NBUF],
                    out_hbm.at[pl.ds(base + j * GCH, GCH)],
                    ssem[j % NBUF])
        for j in range(max(0, nch - NBUF), nch):
            ss[j].wait()

    return gather_k


def _sc_gather(table, idx):
    n_rows = idx.shape[0]
    return _make_sc_gather(
        n_rows, table.shape[0], table.shape[1], table.dtype)(table, idx)


def _expert_body(be_ref, x_ref, wg_ref, wu_ref, wd_ref, o_ref):
    del be_ref
    f = pl.program_id(1)
    x = x_ref[...].astype(jnp.bfloat16)
    gate = lax.dot_general(x, wg_ref[0], (((1,), (1,)), ((), ())),
                           preferred_element_type=jnp.float32)
    up = lax.dot_general(x, wu_ref[0], (((1,), (1,)), ((), ())),
                         preferred_element_type=jnp.float32)
    act = (gate * jax.nn.sigmoid(gate) * up).astype(jnp.bfloat16)
    part = lax.dot_general(act, wd_ref[0], (((1,), (1,)), ((), ())),
                           preferred_element_type=jnp.float32)

    @pl.when(f == 0)
    def _init():
        o_ref[...] = part

    @pl.when(f != 0)
    def _acc():
        o_ref[...] += part


def _expert_mlp(block_expert, x_sorted, wg, wu, wd):
    def _fe(b, f):
        return jnp.where(b % 2 == 0, f, NFB - 1 - f)

    grid_spec = pltpu.PrefetchScalarGridSpec(
        num_scalar_prefetch=1,
        grid=(NBLK, NFB),
        in_specs=[
            pl.BlockSpec((BM, H), lambda b, f, be: (b, 0)),
            pl.BlockSpec((1, BF, H), lambda b, f, be: (be[b], _fe(b, f), 0)),
            pl.BlockSpec((1, BF, H), lambda b, f, be: (be[b], _fe(b, f), 0)),
            pl.BlockSpec((1, H, BF), lambda b, f, be: (be[b], 0, _fe(b, f))),
        ],
        out_specs=pl.BlockSpec((BM, H), lambda b, f, be: (b, 0)),
    )
    return pl.pallas_call(
        _expert_body,
        grid_spec=grid_spec,
        out_shape=jax.ShapeDtypeStruct((S_PAD, H), jnp.float32),
    )(block_expert, x_sorted, wg, wu, wd)


def _shared_body(x_ref, wg_ref, wu_ref, wd_ref, o_ref):
    x = x_ref[...]
    gate = lax.dot_general(x, wg_ref[...], (((1,), (1,)), ((), ())),
                           preferred_element_type=jnp.float32)
    up = lax.dot_general(x, wu_ref[...], (((1,), (1,)), ((), ())),
                         preferred_element_type=jnp.float32)
    act = (gate * jax.nn.sigmoid(gate) * up).astype(jnp.bfloat16)
    o_ref[...] = lax.dot_general(act, wd_ref[...], (((1,), (1,)), ((), ())),
                                 preferred_element_type=jnp.float32)


def _shared_mlp(x16, wg_s, wu_s, wd_s):
    return pl.pallas_call(
        _shared_body,
        grid=(T // BM3,),
        in_specs=[
            pl.BlockSpec((BM3, H), lambda t: (t, 0)),
            pl.BlockSpec((F, H), lambda t: (0, 0)),
            pl.BlockSpec((F, H), lambda t: (0, 0)),
            pl.BlockSpec((H, F), lambda t: (0, 0)),
        ],
        out_specs=pl.BlockSpec((BM3, H), lambda t: (t, 0)),
        out_shape=jax.ShapeDtypeStruct((T, H), jnp.float32),
    )(x16, wg_s, wu_s, wd_s)


def _combine_body(sh_ref, y0_ref, y1_ref, s0_ref, s1_ref, o_ref):
    o_ref[...] = (sh_ref[...] + y0_ref[0] * s0_ref[...]
                  + y1_ref[0] * s1_ref[...])


def _combine(shared, yk, s0, s1):
    return pl.pallas_call(
        _combine_body,
        grid=(T // BM3,),
        in_specs=[
            pl.BlockSpec((BM3, H), lambda t: (t, 0)),
            pl.BlockSpec((1, BM3, H), lambda t: (0, t, 0)),
            pl.BlockSpec((1, BM3, H), lambda t: (1, t, 0)),
            pl.BlockSpec((BM3, 1), lambda t: (t, 0)),
            pl.BlockSpec((BM3, 1), lambda t: (t, 0)),
        ],
        out_specs=pl.BlockSpec((BM3, H), lambda t: (t, 0)),
        out_shape=jax.ShapeDtypeStruct((T, H), jnp.float32),
    )(shared, yk, yk, s0, s1)


def kernel(hidden_states, W_router, Wg_experts, Wu_experts, Wd_experts,
           Wg_shared, Wu_shared, Wd_shared):
    x = hidden_states
    idx, scores = _router(x, W_router)

    flat_e = idx.reshape(-1)
    onehot = (flat_e[:, None] == jnp.arange(E, dtype=jnp.int32))
    onehot = onehot.astype(jnp.int32)
    within = jnp.cumsum(onehot, axis=0) - onehot
    counts = jnp.sum(onehot, axis=0)
    padded = ((counts + BM - 1) // BM) * BM
    poff = jnp.concatenate(
        [jnp.zeros((1,), jnp.int32),
         jnp.cumsum(padded)[:-1].astype(jnp.int32)])
    pos = (poff[flat_e]
           + jnp.take_along_axis(within, flat_e[:, None], axis=1)[:, 0])

    gather_idx = jnp.zeros((S_PAD,), jnp.int32).at[pos].set(
        jnp.arange(TK, dtype=jnp.int32) // K)
    pend = poff + padded
    block_start = jnp.arange(NBLK, dtype=jnp.int32) * BM
    block_expert = jnp.sum(
        (block_start[:, None] >= pend[None, :]).astype(jnp.int32), axis=1)
    block_expert = jnp.minimum(block_expert, E - 1)
    pos_km = pos.reshape(T, K).T.reshape(-1)

    x_sorted = _sc_gather(x, gather_idx)
    shared = _shared_mlp(
        x.astype(jnp.bfloat16),
        Wg_shared.astype(jnp.bfloat16),
        Wu_shared.astype(jnp.bfloat16),
        Wd_shared.astype(jnp.bfloat16))
    y_sorted = _expert_mlp(
        block_expert, x_sorted,
        Wg_experts.astype(jnp.bfloat16),
        Wu_experts.astype(jnp.bfloat16),
        Wd_experts.astype(jnp.bfloat16))
    yk = _sc_gather(y_sorted, pos_km).reshape(K, T, H)

    return _combine(shared, yk, scores[:, 0:1], scores[:, 1:2])


# speedup vs baseline: 1.0458x; 1.0458x over previous
import functools

import jax
import jax.numpy as jnp
from jax import lax
from jax.experimental import pallas as pl
from jax.experimental.pallas import tpu as pltpu
from jax.experimental.pallas import tpu_sc as plsc

E = 8
K = 2
T = 2048
H = 2048
F = 2048
TK = T * K

BM = 256
S_PAD = TK + E * BM
NBLK = S_PAD // BM
BF = 1024
NFB = F // BF
BM3 = 256

SC_NC = 2
SC_NS = 16
SC_NW = SC_NC * SC_NS
GCH = 16


def _router_body(x_ref, wr_ref, idx_ref, score_ref):
    x = x_ref[...]
    wr = wr_ref[...]
    logits = lax.dot_general(x, wr, (((1,), (1,)), ((), ())),
                             preferred_element_type=jnp.float32)
    lane = lax.broadcasted_iota(jnp.int32, (T, E), 1)
    m1 = jnp.max(logits, axis=1, keepdims=True)
    i1 = jnp.min(jnp.where(logits == m1, lane, E), axis=1, keepdims=True)
    masked = jnp.where(lane == i1, -jnp.inf, logits)
    m2 = jnp.max(masked, axis=1, keepdims=True)
    i2 = jnp.min(jnp.where(masked == m2, lane, E), axis=1, keepdims=True)
    idx_ref[:, 0:1] = i1
    idx_ref[:, 1:2] = i2
    score_ref[:, 0:1] = jax.nn.sigmoid(m1)
    score_ref[:, 1:2] = jax.nn.sigmoid(m2)


def _router(x, w_router):
    return pl.pallas_call(
        _router_body,
        out_shape=[
            jax.ShapeDtypeStruct((T, K), jnp.int32),
            jax.ShapeDtypeStruct((T, K), jnp.float32),
        ],
    )(x, w_router)


NBUF = 3
AHEAD = 2


@functools.lru_cache(maxsize=None)
def _make_sc_gather(n_rows, table_rows, width, dtype):
    rpw = n_rows // SC_NW
    nch = rpw // GCH
    mesh = plsc.VectorSubcoreMesh(core_axis_name="c", subcore_axis_name="s")

    @functools.partial(
        pl.kernel,
        mesh=mesh,
        out_type=jax.ShapeDtypeStruct((n_rows, width), dtype),
        scratch_types=[
            pltpu.VMEM((rpw,), jnp.int32),
            *[pltpu.VMEM((GCH, width), dtype) for _ in range(NBUF)],
            *[pltpu.SemaphoreType.DMA for _ in range(2 * NBUF)],
        ],
    )
    def gather_k(table_hbm, idx_hbm, out_hbm, idx_all, *bufs_sems):
        bufs = bufs_sems[:NBUF]
        gsem = bufs_sems[NBUF:2 * NBUF]
        del bufs_sems
        wid = lax.axis_index("s") * SC_NC + lax.axis_index("c")
        base = wid * rpw
        pltpu.sync_copy(idx_hbm.at[pl.ds(base, rpw)], idx_all)
        for i in range(nch):
            pltpu.async_copy(
                table_hbm.at[idx_all.at[pl.ds(i * GCH, GCH)]],
                bufs[0], gsem[0]).wait()
            pltpu.sync_copy(bufs[0], out_hbm.at[pl.ds(base + i * GCH, GCH)])

    return gather_k


def _sc_gather(table, idx):
    n_rows = idx.shape[0]
    return _make_sc_gather(
        n_rows, table.shape[0], table.shape[1], table.dtype)(table, idx)


def _expert_body(be_ref, x_ref, wg_ref, wu_ref, wd_ref, o_ref):
    del be_ref
    f = pl.program_id(1)
    x = x_ref[...].astype(jnp.bfloat16)
    gate = lax.dot_general(x, wg_ref[0], (((1,), (1,)), ((), ())),
                           preferred_element_type=jnp.float32)
    up = lax.dot_general(x, wu_ref[0], (((1,), (1,)), ((), ())),
                         preferred_element_type=jnp.float32)
    act = (gate * jax.nn.sigmoid(gate) * up).astype(jnp.bfloat16)
    part = lax.dot_general(act, wd_ref[0], (((1,), (1,)), ((), ())),
                           preferred_element_type=jnp.float32)

    @pl.when(f == 0)
    def _init():
        o_ref[...] = part

    @pl.when(f != 0)
    def _acc():
        o_ref[...] += part


def _expert_mlp(block_expert, x_sorted, wg, wu, wd):
    def _fe(b, f):
        return jnp.where(b % 2 == 0, f, NFB - 1 - f)

    grid_spec = pltpu.PrefetchScalarGridSpec(
        num_scalar_prefetch=1,
        grid=(NBLK, NFB),
        in_specs=[
            pl.BlockSpec((BM, H), lambda b, f, be: (b, 0)),
            pl.BlockSpec((1, BF, H), lambda b, f, be: (be[b], _fe(b, f), 0)),
            pl.BlockSpec((1, BF, H), lambda b, f, be: (be[b], _fe(b, f), 0)),
            pl.BlockSpec((1, H, BF), lambda b, f, be: (be[b], 0, _fe(b, f))),
        ],
        out_specs=pl.BlockSpec((BM, H), lambda b, f, be: (b, 0)),
    )
    return pl.pallas_call(
        _expert_body,
        grid_spec=grid_spec,
        out_shape=jax.ShapeDtypeStruct((S_PAD, H), jnp.float32),
    )(block_expert, x_sorted, wg, wu, wd)


def _combine_body(x_ref, wg_ref, wu_ref, wd_ref, y0_ref, y1_ref,
                  s0_ref, s1_ref, o_ref):
    x = x_ref[...].astype(jnp.bfloat16)
    gate = lax.dot_general(x, wg_ref[...], (((1,), (1,)), ((), ())),
                           preferred_element_type=jnp.float32)
    up = lax.dot_general(x, wu_ref[...], (((1,), (1,)), ((), ())),
                         preferred_element_type=jnp.float32)
    act = (gate * jax.nn.sigmoid(gate) * up).astype(jnp.bfloat16)
    shared = lax.dot_general(act, wd_ref[...], (((1,), (1,)), ((), ())),
                             preferred_element_type=jnp.float32)
    o_ref[...] = shared + y0_ref[0] * s0_ref[...] + y1_ref[0] * s1_ref[...]


def _combine(x, wg_s, wu_s, wd_s, yk, s0, s1):
    return pl.pallas_call(
        _combine_body,
        grid=(T // BM3,),
        in_specs=[
            pl.BlockSpec((BM3, H), lambda t: (t, 0)),
            pl.BlockSpec((F, H), lambda t: (0, 0)),
            pl.BlockSpec((F, H), lambda t: (0, 0)),
            pl.BlockSpec((H, F), lambda t: (0, 0)),
            pl.BlockSpec((1, BM3, H), lambda t: (0, t, 0)),
            pl.BlockSpec((1, BM3, H), lambda t: (1, t, 0)),
            pl.BlockSpec((BM3, 1), lambda t: (t, 0)),
            pl.BlockSpec((BM3, 1), lambda t: (t, 0)),
        ],
        out_specs=pl.BlockSpec((BM3, H), lambda t: (t, 0)),
        out_shape=jax.ShapeDtypeStruct((T, H), jnp.float32),
    )(x, wg_s, wu_s, wd_s, yk, yk, s0, s1)


def kernel(hidden_states, W_router, Wg_experts, Wu_experts, Wd_experts,
           Wg_shared, Wu_shared, Wd_shared):
    x = hidden_states
    idx, scores = _router(x, W_router)

    flat_e = idx.reshape(-1)
    onehot = (flat_e[:, None] == jnp.arange(E, dtype=jnp.int32))
    onehot = onehot.astype(jnp.int32)
    within = jnp.cumsum(onehot, axis=0) - onehot
    counts = jnp.sum(onehot, axis=0)
    padded = ((counts + BM - 1) // BM) * BM
    poff = jnp.concatenate(
        [jnp.zeros((1,), jnp.int32),
         jnp.cumsum(padded)[:-1].astype(jnp.int32)])
    pos = (poff[flat_e]
           + jnp.take_along_axis(within, flat_e[:, None], axis=1)[:, 0])

    gather_idx = jnp.zeros((S_PAD,), jnp.int32).at[pos].set(
        jnp.arange(TK, dtype=jnp.int32) // K)
    pend = poff + padded
    block_start = jnp.arange(NBLK, dtype=jnp.int32) * BM
    block_expert = jnp.sum(
        (block_start[:, None] >= pend[None, :]).astype(jnp.int32), axis=1)
    block_expert = jnp.minimum(block_expert, E - 1)
    pos_km = pos.reshape(T, K).T.reshape(-1)

    x_sorted = _sc_gather(x, gather_idx)
    y_sorted = _expert_mlp(
        block_expert, x_sorted,
        Wg_experts.astype(jnp.bfloat16),
        Wu_experts.astype(jnp.bfloat16),
        Wd_experts.astype(jnp.bfloat16))
    yk = _sc_gather(y_sorted, pos_km).reshape(K, T, H)

    return _combine(
        x,
        Wg_shared.astype(jnp.bfloat16),
        Wu_shared.astype(jnp.bfloat16),
        Wd_shared.astype(jnp.bfloat16),
        yk, scores[:, 0:1], scores[:, 1:2])
